# baseline (speedup 1.0000x reference)
import functools

import jax
import jax.numpy as jnp
from jax import lax
from jax.experimental import pallas as pl
from jax.experimental.pallas import tpu as pltpu
from jax.experimental.pallas import tpu_sc as plsc

N = 10000
E = 160000
D = 256
DH = 128
DE = 16

NS = 16
EPT = E // NS
B = 80
NB = EPT // B
NP = 10240
ROWS_PT = NP // NS
CH = 16
NCH = ROWS_PT // CH
NCK = B // 16
TRASH = N


def _mm_relu_split(x, wt, b, bm):
    M, K = x.shape
    dout = wt.shape[1]

    def body(x_ref, wt_ref, b_ref, oa_ref, ob_ref):
        y = jnp.dot(x_ref[...], wt_ref[...], preferred_element_type=jnp.float32)
        y = jnp.maximum(y + b_ref[...], 0.0)
        oa_ref[...] = y[:, :DH]
        ob_ref[...] = y[:, DH:]

    return pl.pallas_call(
        body,
        grid=(M // bm,),
        in_specs=[
            pl.BlockSpec((bm, K), lambda i: (i, 0)),
            pl.BlockSpec((K, dout), lambda i: (0, 0)),
            pl.BlockSpec((1, dout), lambda i: (0, 0)),
        ],
        out_specs=[
            pl.BlockSpec((bm, DH), lambda i: (i, 0)),
            pl.BlockSpec((bm, DH), lambda i: (i, 0)),
        ],
        out_shape=[jax.ShapeDtypeStruct((M, DH), jnp.float32)] * 2,
    )(x, wt, b.reshape(1, dout))


def _compute_mp(dv_ref, lanes):
    dvs = [dv_ref[pl.ds(ci * 16, 16)] for ci in range(NCK)]
    poss = [lanes + ci * 16 for ci in range(NCK)]
    mps = list(poss)
    for ci in range(NCK):
        def body(l, mps_t, ci=ci):
            p = ci * 16 + l
            idx = jnp.full((16,), l, jnp.int32)
            vs = dvs[ci].at[idx].get(mode="promise_in_bounds")
            out = []
            for cj in range(NCK):
                upd = (dvs[cj] == vs) & (mps_t[cj] == poss[cj]) & (poss[cj] > p)
                out.append(jnp.where(upd, p, mps_t[cj]))
            return tuple(out)

        mps = list(lax.fori_loop(0, 16, body, tuple(mps)))
    return dvs, poss, mps


def _merge_scan(mps, fn, lanes):
    for cj in range(NCK):
        def body(l, carry, cj=cj):
            p = cj * 16 + l
            idx = jnp.full((16,), l, jnp.int32)
            jg = mps[cj].at[idx].get(mode="promise_in_bounds")
            j = jnp.where(lanes == 0, jg, 0)[0]

            @pl.when(j != p)
            def _():
                fn(p, j)

            return carry

        lax.fori_loop(0, 16, body, 0)


def _dedup(dv_ref, merge_fn, lanes):
    dvs, poss, mps = _compute_mp(dv_ref, lanes)
    _merge_scan(mps, merge_fn, lanes)
    for cj in range(NCK):
        dv_ref[pl.ds(cj * 16, 16)] = jnp.where(
            mps[cj] != poss[cj], TRASH, dvs[cj])
    return mps


def _zero_rowbuf(rowbuf):
    zv = jnp.zeros((16,), jnp.float32)
    for i in range(CH):
        for j in range(DH // 16):
            rowbuf[i, pl.ds(j * 16, 16)] = zv


def _zinit_loop(accm, rowbuf, r0):
    def zinit(k, carry):
        pltpu.sync_copy(rowbuf, accm.at[pl.ds(r0 + k * CH, CH)])
        return carry

    lax.fori_loop(0, NCH, zinit, 0)


def _flush_to(accm, rowbuf, r0, out_h):
    def flush(k, carry):
        rr = r0 + k * CH
        pltpu.sync_copy(accm.at[pl.ds(rr, CH)], rowbuf)
        pltpu.sync_copy(rowbuf, out_h.at[pl.ds(rr, CH)])
        return carry

    lax.fori_loop(0, NCH, flush, 0)


def _make_sc_counts():
    mesh = plsc.VectorSubcoreMesh(core_axis_name="c", subcore_axis_name="s")

    @functools.partial(
        pl.kernel,
        out_type=[
            jax.ShapeDtypeStruct((NP, DH), jnp.float32),
            jax.ShapeDtypeStruct((NP, DH), jnp.float32),
        ],
        mesh=mesh,
        scratch_types=[
            pltpu.VMEM_SHARED((NP, DH), jnp.float32),
            pltpu.VMEM((B,), jnp.int32),
            pltpu.VMEM((B, DH), jnp.float32),
            pltpu.VMEM((CH, DH), jnp.float32),
        ],
    )
    def sc_counts(dst_h, cnta_h, cntb_h, accm, dst_v, wrows, rowbuf):
        c = lax.axis_index("c")
        s = lax.axis_index("s")
        r0 = s * ROWS_PT
        lanes = lax.iota(jnp.int32, 16)

        _zero_rowbuf(rowbuf)
        _zinit_loop(accm, rowbuf, r0)
        ov = jnp.ones((16,), jnp.float32)

        def ones_row(i, _):
            for j in range(DH // 16):
                wrows[i, pl.ds(j * 16, 16)] = ov
            return 0

        lax.fori_loop(0, B, ones_row, 0)
        plsc.subcore_barrier()

        start = jnp.where(c == 0, s * 63, 1008 + s * 62)
        nb_w = jnp.where(c == 0, 63, 62)

        def cbatch(bi, carry):
            base = (start + bi) * B
            pltpu.sync_copy(dst_h.at[pl.ds(base, B)], dst_v)

            def merge_ones(i, j):
                sl = pl.ds(0, 16)
                wrows[j, sl] = wrows[j, sl] + 1.0

            mps = _dedup(dst_v, merge_ones, lanes)
            pltpu.sync_copy(wrows, accm.at[dst_v], add=True)

            def restore_ones(i, j):
                wrows[j, pl.ds(0, 16)] = jnp.ones((16,), jnp.float32)

            _merge_scan(mps, restore_ones, lanes)
            return carry

        lax.fori_loop(0, nb_w, cbatch, 0)
        plsc.subcore_barrier()

        @pl.when(c == 0)
        def _():
            _flush_to(accm, rowbuf, r0, cnta_h)

        @pl.when(c == 1)
        def _():
            _flush_to(accm, rowbuf, r0, cntb_h)

    return sc_counts


def _make_sc_msum():
    mesh = plsc.VectorSubcoreMesh(core_axis_name="c", subcore_axis_name="s")

    @functools.partial(
        pl.kernel,
        out_type=[
            jax.ShapeDtypeStruct((NP, DH), jnp.float32),
            jax.ShapeDtypeStruct((NP, DH), jnp.float32),
        ],
        mesh=mesh,
        scratch_types=[
            pltpu.VMEM_SHARED((NP, DH), jnp.float32),
            pltpu.VMEM((B,), jnp.int32),
            pltpu.VMEM((B,), jnp.int32),
            pltpu.VMEM((B, DH), jnp.float32),
            pltpu.VMEM((B, DH), jnp.float32),
            pltpu.VMEM((CH, DH), jnp.float32),
            pltpu.VMEM((B,), jnp.int32),
            pltpu.VMEM((B,), jnp.int32),
            pltpu.VMEM((B, DH), jnp.float32),
            pltpu.SemaphoreType.DMA,
            pltpu.SemaphoreType.DMA,
            pltpu.SemaphoreType.DMA,
        ],
    )
    def sc_msum(src_h, dst_h, wa_h, wb_h, ha_h, hb_h,
                msuma_h, msumb_h,
                accm, src_v, dst_v, wrows, hrows, rowbuf,
                src_v2, dst_v2, hrows2, sem, sema, semb):
        c = lax.axis_index("c")
        s = lax.axis_index("s")
        r0 = s * ROWS_PT
        e0 = s * EPT
        lanes = lax.iota(jnp.int32, 16)

        def run(w_h, h_h, msum_h):
            _zero_rowbuf(rowbuf)
            _zinit_loop(accm, rowbuf, r0)
            plsc.subcore_barrier()

            bufs = ((src_v, dst_v, hrows, sema), (src_v2, dst_v2, hrows2, semb))

            def issue(bi, sv, dv, hv, sm):
                base = e0 + bi * B
                pltpu.async_copy(src_h.at[pl.ds(base, B)], sv, sm)
                pltpu.async_copy(dst_h.at[pl.ds(base, B)], dv, sm)
                pltpu.async_copy(h_h.at[pl.ds(base, B)], hv, sm)

            def drain(sv, dv, hv, sm):
                pltpu.make_async_copy(src_h.at[pl.ds(0, B)], sv, sm).wait()
                pltpu.make_async_copy(dst_h.at[pl.ds(0, B)], dv, sm).wait()
                pltpu.make_async_copy(h_h.at[pl.ds(0, B)], hv, sm).wait()

            def process(sv, dv, hv):
                pltpu.async_copy(w_h.at[sv], wrows, sem).wait()

                def mul_row(i, _):
                    for j in range(DH // 16):
                        sl = pl.ds(j * 16, 16)
                        wrows[i, sl] = wrows[i, sl] * hv[i, sl]
                    return 0

                lax.fori_loop(0, B, mul_row, 0)

                def merge_rows(i, j):
                    for k in range(DH // 16):
                        sl = pl.ds(k * 16, 16)
                        wrows[j, sl] = wrows[j, sl] + wrows[i, sl]

                _dedup(dv, merge_rows, lanes)
                pltpu.sync_copy(wrows, accm.at[dv], add=True)

            issue(0, *bufs[0])

            def pairbody(k, carry):
                for par in range(2):
                    g = 2 * k + par
                    sv, dv, hv, sm = bufs[par]
                    drain(sv, dv, hv, sm)
                    issue(g + 1, *bufs[1 - par])
                    process(sv, dv, hv)
                return carry

            lax.fori_loop(0, NB // 2, pairbody, 0)
            drain(*bufs[0])
            process(bufs[0][0], bufs[0][1], bufs[0][2])
            plsc.subcore_barrier()
            _flush_to(accm, rowbuf, r0, msum_h)

        @pl.when(c == 0)
        def _():
            run(wa_h, ha_h, msuma_h)

        @pl.when(c == 1)
        def _():
            run(wb_h, hb_h, msumb_h)

    return sc_msum


_sc_counts = _make_sc_counts()
_sc_msum = _make_sc_msum()


def _divide(msuma, msumb, cnta, cntb):
    bn = 1000

    def body(a_ref, b_ref, ca_ref, cb_ref, o_ref):
        cnt = ca_ref[:, 0:1] + cb_ref[:, 0:1]
        inv = 1.0 / jnp.maximum(cnt, 1.0)
        o_ref[:, :DH] = a_ref[...] * inv
        o_ref[:, DH:] = b_ref[...] * inv

    return pl.pallas_call(
        body,
        grid=(N // bn,),
        in_specs=[
            pl.BlockSpec((bn, DH), lambda i: (i, 0)),
            pl.BlockSpec((bn, DH), lambda i: (i, 0)),
            pl.BlockSpec((bn, DH), lambda i: (i, 0)),
            pl.BlockSpec((bn, DH), lambda i: (i, 0)),
        ],
        out_specs=pl.BlockSpec((bn, D), lambda i: (i, 0)),
        out_shape=jax.ShapeDtypeStruct((N, D), jnp.float32),
    )(msuma, msumb, cnta, cntb)


def kernel(feature, efeature, edge_index, nweight, nbias, eweight, ebias):
    src = edge_index[0]
    dst = edge_index[1]
    cnta, cntb = _sc_counts(dst)
    wa, wb = _mm_relu_split(feature, nweight.T, nbias, bm=1000)
    ha, hb = _mm_relu_split(efeature, eweight.T, ebias, bm=2000)
    msuma, msumb = _sc_msum(src, dst, wa, wb, ha, hb)
    return _divide(msuma, msumb, cnta, cntb)


# speedup vs baseline: 1.0053x; 1.0053x over previous
import functools

import jax
import jax.numpy as jnp
from jax import lax
from jax.experimental import pallas as pl
from jax.experimental.pallas import tpu as pltpu
from jax.experimental.pallas import tpu_sc as plsc

N = 10000
E = 160000
D = 256
DH = 128
DE = 16

NS = 16
EPT = E // NS
B = 80
NB = EPT // B
NP = 10240
ROWS_PT = NP // NS
CH = 16
NCH = ROWS_PT // CH
NCK = B // 16
TRASH = N

def _mm_relu_split(x, wt, b, bm):
    M, K = x.shape
    dout = wt.shape[1]

    def body(x_ref, wt_ref, b_ref, oa_ref, ob_ref):
        y = jnp.dot(x_ref[...], wt_ref[...], preferred_element_type=jnp.float32)
        y = jnp.maximum(y + b_ref[...], 0.0)
        oa_ref[...] = y[:, :DH]
        ob_ref[...] = y[:, DH:]

    return pl.pallas_call(
        body,
        grid=(M // bm,),
        in_specs=[
            pl.BlockSpec((bm, K), lambda i: (i, 0)),
            pl.BlockSpec((K, dout), lambda i: (0, 0)),
            pl.BlockSpec((1, dout), lambda i: (0, 0)),
        ],
        out_specs=[
            pl.BlockSpec((bm, DH), lambda i: (i, 0)),
            pl.BlockSpec((bm, DH), lambda i: (i, 0)),
        ],
        out_shape=[jax.ShapeDtypeStruct((M, DH), jnp.float32)] * 2,
    )(x, wt, b.reshape(1, dout))


def _make_sc_scatter():
    mesh = plsc.VectorSubcoreMesh(core_axis_name="c", subcore_axis_name="s")

    @functools.partial(
        pl.kernel,
        out_type=[
            jax.ShapeDtypeStruct((NP, DH), jnp.float32),
            jax.ShapeDtypeStruct((NP, DH), jnp.float32),
            jax.ShapeDtypeStruct((NP, DH), jnp.float32),
            jax.ShapeDtypeStruct((NP, DH), jnp.float32),
        ],
        mesh=mesh,
        scratch_types=[
            pltpu.VMEM_SHARED((NP, DH), jnp.float32),
            pltpu.VMEM((B,), jnp.int32),
            pltpu.VMEM((B,), jnp.int32),
            pltpu.VMEM((B, DH), jnp.float32),
            pltpu.VMEM((B, DH), jnp.float32),
            pltpu.VMEM((CH, DH), jnp.float32),
            pltpu.VMEM((B,), jnp.int32),
            pltpu.VMEM((B,), jnp.int32),
            pltpu.VMEM((B, DH), jnp.float32),
            pltpu.SemaphoreType.DMA,
            pltpu.SemaphoreType.DMA,
            pltpu.SemaphoreType.DMA,
        ],
    )
    def sc_scatter(src_h, dst_h, wa_h, wb_h, ha_h, hb_h,
                   msuma_h, msumb_h, cnta_h, cntb_h,
                   accm, src_v, dst_v, wrows, hrows, rowbuf,
                   src_v2, dst_v2, hrows2, sem, sema, semb):
        c = lax.axis_index("c")
        s = lax.axis_index("s")
        r0 = s * ROWS_PT
        e0 = s * EPT
        lanes = lax.iota(jnp.int32, 16)

        def compute_mp(dv_ref):
            dvs = [dv_ref[pl.ds(ci * 16, 16)] for ci in range(NCK)]
            poss = [lanes + ci * 16 for ci in range(NCK)]
            mps = list(poss)
            for ci in range(NCK):
                def body(l, mps_t, ci=ci):
                    p = ci * 16 + l
                    idx = jnp.full((16,), l, jnp.int32)
                    vs = dvs[ci].at[idx].get(mode="promise_in_bounds")
                    out = []
                    for cj in range(NCK):
                        upd = ((dvs[cj] == vs) & (mps_t[cj] == poss[cj])
                               & (poss[cj] > p))
                        out.append(jnp.where(upd, p, mps_t[cj]))
                    return tuple(out)

                mps = list(lax.fori_loop(0, 16, body, tuple(mps)))
            return dvs, poss, mps

        def merge_scan(mps, fn):
            for cj in range(NCK):
                def body(l, carry, cj=cj):
                    p = cj * 16 + l
                    idx = jnp.full((16,), l, jnp.int32)
                    jg = mps[cj].at[idx].get(mode="promise_in_bounds")
                    j = jnp.where(lanes == 0, jg, 0)[0]

                    @pl.when(j != p)
                    def _():
                        fn(p, j)

                    return carry

                lax.fori_loop(0, 16, body, 0)

        def dedup(dv_ref, merge_fn):
            dvs, poss, mps = compute_mp(dv_ref)
            merge_scan(mps, merge_fn)
            for cj in range(NCK):
                dv_ref[pl.ds(cj * 16, 16)] = jnp.where(
                    mps[cj] != poss[cj], TRASH, dvs[cj])
            return mps

        def run(w_h, h_h, msum_h, cnt_h):
            zv = jnp.zeros((16,), jnp.float32)
            for i in range(CH):
                for j in range(DH // 16):
                    rowbuf[i, pl.ds(j * 16, 16)] = zv

            def zinit(k, carry):
                pltpu.sync_copy(rowbuf, accm.at[pl.ds(r0 + k * CH, CH)])
                return carry

            def flush_to(out_h):
                def flush(k, carry):
                    rr = r0 + k * CH
                    pltpu.sync_copy(accm.at[pl.ds(rr, CH)], rowbuf)
                    pltpu.sync_copy(rowbuf, out_h.at[pl.ds(rr, CH)])
                    return carry

                lax.fori_loop(0, NCH, flush, 0)

            lax.fori_loop(0, NCH, zinit, 0)
            plsc.subcore_barrier()

            bufs = ((src_v, dst_v, hrows, sema), (src_v2, dst_v2, hrows2, semb))

            def issue(bi, sv, dv, hv, sm):
                base = e0 + bi * B
                pltpu.async_copy(src_h.at[pl.ds(base, B)], sv, sm)
                pltpu.async_copy(dst_h.at[pl.ds(base, B)], dv, sm)
                pltpu.async_copy(h_h.at[pl.ds(base, B)], hv, sm)

            def drain(sv, dv, hv, sm):
                pltpu.make_async_copy(src_h.at[pl.ds(0, B)], sv, sm).wait()
                pltpu.make_async_copy(dst_h.at[pl.ds(0, B)], dv, sm).wait()
                pltpu.make_async_copy(h_h.at[pl.ds(0, B)], hv, sm).wait()

            def process(sv, dv, hv):
                pltpu.async_copy(w_h.at[sv], wrows, sem).wait()

                def mul_row(i, _):
                    for j in range(DH // 16):
                        sl = pl.ds(j * 16, 16)
                        wrows[i, sl] = wrows[i, sl] * hv[i, sl]
                    return 0

                lax.fori_loop(0, B, mul_row, 0)

                def merge_rows(i, j):
                    for k in range(DH // 16):
                        sl = pl.ds(k * 16, 16)
                        wrows[j, sl] = wrows[j, sl] + wrows[i, sl]

                dedup(dv, merge_rows)
                pltpu.sync_copy(wrows, accm.at[dv], add=True)

            issue(0, *bufs[0])

            def pairbody(k, carry):
                for par in range(2):
                    g = 2 * k + par
                    sv, dv, hv, sm = bufs[par]
                    drain(sv, dv, hv, sm)
                    issue(g + 1, *bufs[1 - par])
                    process(sv, dv, hv)
                return carry

            lax.fori_loop(0, NB // 2, pairbody, 0)
            drain(*bufs[0])
            process(bufs[0][0], bufs[0][1], bufs[0][2])
            plsc.subcore_barrier()
            flush_to(msum_h)

            for i in range(CH):
                for j in range(DH // 16):
                    rowbuf[i, pl.ds(j * 16, 16)] = zv
            lax.fori_loop(0, NCH, zinit, 0)
            ov = jnp.ones((16,), jnp.float32)

            def ones_row(i, _):
                for j in range(DH // 16):
                    wrows[i, pl.ds(j * 16, 16)] = ov
                return 0

            lax.fori_loop(0, B, ones_row, 0)
            plsc.subcore_barrier()

            start = jnp.where(c == 0, s * 63, 1008 + s * 62)
            nb_w = jnp.where(c == 0, 63, 62)

            def cbatch(bi, carry):
                base = (start + bi) * B
                pltpu.sync_copy(dst_h.at[pl.ds(base, B)], dst_v)

                def merge_ones(i, j):
                    sl = pl.ds(0, 16)
                    wrows[j, sl] = wrows[j, sl] + 1.0

                mps = dedup(dst_v, merge_ones)
                pltpu.sync_copy(wrows, accm.at[dst_v], add=True)

                def restore_ones(i, j):
                    wrows[j, pl.ds(0, 16)] = jnp.ones((16,), jnp.float32)

                merge_scan(mps, restore_ones)
                return carry

            lax.fori_loop(0, nb_w, cbatch, 0)
            plsc.subcore_barrier()
            flush_to(cnt_h)

        @pl.when(c == 0)
        def _():
            run(wa_h, ha_h, msuma_h, cnta_h)

        @pl.when(c == 1)
        def _():
            run(wb_h, hb_h, msumb_h, cntb_h)

    return sc_scatter


_sc_scatter = _make_sc_scatter()


def _divide(msuma, msumb, cnta, cntb):
    bn = 1000

    def body(a_ref, b_ref, ca_ref, cb_ref, o_ref):
        cnt = ca_ref[:, 0:1] + cb_ref[:, 0:1]
        inv = 1.0 / jnp.maximum(cnt, 1.0)
        o_ref[:, :DH] = a_ref[...] * inv
        o_ref[:, DH:] = b_ref[...] * inv

    return pl.pallas_call(
        body,
        grid=(N // bn,),
        in_specs=[
            pl.BlockSpec((bn, DH), lambda i: (i, 0)),
            pl.BlockSpec((bn, DH), lambda i: (i, 0)),
            pl.BlockSpec((bn, DH), lambda i: (i, 0)),
            pl.BlockSpec((bn, DH), lambda i: (i, 0)),
        ],
        out_specs=pl.BlockSpec((bn, D), lambda i: (i, 0)),
        out_shape=jax.ShapeDtypeStruct((N, D), jnp.float32),
    )(msuma, msumb, cnta, cntb)


def kernel(feature, efeature, edge_index, nweight, nbias, eweight, ebias):
    src = edge_index[0]
    dst = edge_index[1]
    wa, wb = _mm_relu_split(feature, nweight.T, nbias, bm=1000)
    ha, hb = _mm_relu_split(efeature, eweight.T, ebias, bm=2000)
    msuma, msumb, cnta, cntb = _sc_scatter(src, dst, wa, wb, ha, hb)
    return _divide(msuma, msumb, cnta, cntb)
